# baseline (speedup 1.0000x reference)
import jax
import jax.numpy as jnp
from jax.experimental import pallas as pl

_BLOCK = 4000


def _mlp_block(x_ref, rbf_ref, wrbf_ref, wup_ref, w1_ref, b1_ref, w2_ref,
               b2_ref, wout_ref, o_ref):
    h = jnp.dot(rbf_ref[...], wrbf_ref[...],
                preferred_element_type=jnp.float32) * x_ref[...]
    o = jnp.dot(h, wup_ref[...], preferred_element_type=jnp.float32)
    o = jax.nn.silu(jnp.dot(o, w1_ref[...],
                            preferred_element_type=jnp.float32) + b1_ref[...])
    o = jax.nn.silu(jnp.dot(o, w2_ref[...],
                            preferred_element_type=jnp.float32) + b2_ref[...])
    o_ref[...] = jnp.dot(o, wout_ref[...], preferred_element_type=jnp.float32)


def kernel(x, rbf, i, num_nodes, W_rbf, W_up, W1, b1, W2, b2, W_out):
    del i, num_nodes
    E, H = x.shape
    R = rbf.shape[1]
    D = W_up.shape[1]
    b1 = b1.reshape(1, D)
    b2 = b2.reshape(1, D)

    grid = (E // _BLOCK,)
    row_spec = lambda shape: pl.BlockSpec(shape, lambda m: (m, 0))
    rep_spec = lambda shape: pl.BlockSpec(shape, lambda m: (0, 0))

    return pl.pallas_call(
        _mlp_block,
        grid=grid,
        in_specs=[
            row_spec((_BLOCK, H)),
            row_spec((_BLOCK, R)),
            rep_spec((R, H)),
            rep_spec((H, D)),
            rep_spec((D, D)),
            rep_spec((1, D)),
            rep_spec((D, D)),
            rep_spec((1, D)),
            rep_spec((D, 1)),
        ],
        out_specs=row_spec((_BLOCK, 1)),
        out_shape=jax.ShapeDtypeStruct((E, 1), jnp.float32),
    )(x, rbf, W_rbf, W_up, W1, b1, W2, b2, W_out)


# speedup vs baseline: 1.0883x; 1.0883x over previous
import jax
import jax.numpy as jnp
from jax.experimental import pallas as pl

_BLOCK = 8000


def _mlp_block(x_ref, rbf_ref, wrbf_ref, wup_ref, w1_ref, b1_ref, w2_ref,
               b2_ref, wout_ref, o_ref):
    h = jnp.dot(rbf_ref[...], wrbf_ref[...],
                preferred_element_type=jnp.float32) * x_ref[...]
    o = jnp.dot(h, wup_ref[...], preferred_element_type=jnp.float32)
    o = jax.nn.silu(jnp.dot(o, w1_ref[...],
                            preferred_element_type=jnp.float32) + b1_ref[...])
    o = jax.nn.silu(jnp.dot(o, w2_ref[...],
                            preferred_element_type=jnp.float32) + b2_ref[...])
    o_ref[...] = jnp.dot(o, wout_ref[...], preferred_element_type=jnp.float32)


def kernel(x, rbf, i, num_nodes, W_rbf, W_up, W1, b1, W2, b2, W_out):
    del i, num_nodes
    E, H = x.shape
    R = rbf.shape[1]
    D = W_up.shape[1]
    b1 = b1.reshape(1, D)
    b2 = b2.reshape(1, D)

    grid = (E // _BLOCK,)
    row_spec = lambda shape: pl.BlockSpec(shape, lambda m: (m, 0))
    rep_spec = lambda shape: pl.BlockSpec(shape, lambda m: (0, 0))

    return pl.pallas_call(
        _mlp_block,
        grid=grid,
        in_specs=[
            row_spec((_BLOCK, H)),
            row_spec((_BLOCK, R)),
            rep_spec((R, H)),
            rep_spec((H, D)),
            rep_spec((D, D)),
            rep_spec((1, D)),
            rep_spec((D, D)),
            rep_spec((1, D)),
            rep_spec((D, 1)),
        ],
        out_specs=row_spec((_BLOCK, 1)),
        out_shape=jax.ShapeDtypeStruct((E, 1), jnp.float32),
    )(x, rbf, W_rbf, W_up, W1, b1, W2, b2, W_out)
